# baseline (speedup 1.0000x reference)
import functools

import jax
import jax.numpy as jnp
from jax import lax
from jax.experimental import pallas as pl
from jax.experimental.pallas import tpu as pltpu
from jax.experimental.pallas import tpu_sc as plsc

_N = 10000
_E = 320000
_B = 128
_NPAD = 10112
_CH = 128
_CPI = 8
_TILES = 16
_ITERS = 20
_CHUNKS = _CPI * _ITERS
_EPAD = _TILES * _CHUNKS * _CH
_ROWS = _NPAD // _TILES
_ERB = _EPAD // _CH
_NB = _NPAD // 128
_PCPI = 4
_WCH = _ERB // 32
_PH = _WCH // _PCPI


def _sc_degree(dst2, ones16, zeros16):
    mesh = plsc.VectorSubcoreMesh(core_axis_name="c", subcore_axis_name="s")

    @functools.partial(
        pl.kernel, mesh=mesh,
        compiler_params=pltpu.CompilerParams(use_tc_tiling_on_sc=False),
        out_type=jax.ShapeDtypeStruct((2 * _NPAD, 16), jnp.float32),
        scratch_types=[
            pltpu.VMEM((_CPI, _CH), jnp.int32),
            pltpu.VMEM((_CH, 16), jnp.float32),
            pltpu.VMEM_SHARED((_NPAD, 16), jnp.float32),
        ],
    )
    def deg_kernel(dst_hbm, ones_hbm, z_hbm, out_hbm, didx, ones_v, acc):
        c = lax.axis_index("c")
        s = lax.axis_index("s")
        r0 = s * _ROWS
        pltpu.sync_copy(z_hbm.at[pl.ds(r0, _ROWS)], acc.at[pl.ds(r0, _ROWS)])
        pltpu.sync_copy(ones_hbm, ones_v)
        plsc.subcore_barrier()
        rbase = c * _ERB + s * _CHUNKS

        def body(i, carry):
            pltpu.sync_copy(dst_hbm.at[pl.ds(rbase + i * _CPI, _CPI)], didx)
            for j in range(_CPI):
                pltpu.sync_copy(ones_v, acc.at[didx.at[j]], add=True)
            return carry

        lax.fori_loop(0, _ITERS, body, 0)
        plsc.subcore_barrier()
        pltpu.sync_copy(acc.at[pl.ds(r0, _ROWS)],
                        out_hbm.at[pl.ds(c * _NPAD + r0, _ROWS)])

    return deg_kernel(dst2, ones16, zeros16)


def _sc_edge_scatter(y, src, dst, zeros64):
    mesh = plsc.VectorSubcoreMesh(core_axis_name="c", subcore_axis_name="s")

    @functools.partial(
        pl.kernel, mesh=mesh,
        compiler_params=pltpu.CompilerParams(use_tc_tiling_on_sc=False),
        out_type=jax.ShapeDtypeStruct((2 * _NPAD, 64), jnp.float32),
        scratch_types=[
            pltpu.VMEM((2, _PCPI, _CH), jnp.int32),
            pltpu.VMEM((2, _PCPI, _CH), jnp.int32),
            pltpu.VMEM((2, _PCPI * _CH, 64), jnp.float32),
            pltpu.VMEM_SHARED((_NPAD, 64), jnp.float32),
            pltpu.SemaphoreType.DMA,
            pltpu.SemaphoreType.DMA,
        ],
    )
    def edge_kernel(y_hbm, src_hbm, dst_hbm, z_hbm, out_hbm,
                    sidx, didx, rows, acc, sem_g, sem_s):
        c = lax.axis_index("c")
        s = lax.axis_index("s")
        r0 = s * _ROWS
        pltpu.sync_copy(z_hbm.at[pl.ds(r0, _ROWS)], acc.at[pl.ds(r0, _ROWS)])
        plsc.subcore_barrier()
        rbase = (s * 2 + c) * _WCH

        def load_idx(p, b):
            off = rbase + p * _PCPI
            pltpu.sync_copy(src_hbm.at[pl.ds(off, _PCPI)], sidx.at[b])
            pltpu.sync_copy(dst_hbm.at[pl.ds(off, _PCPI)], didx.at[b])

        def gathers(b):
            return [pltpu.make_async_copy(
                        y_hbm.at[sidx.at[b, j]],
                        rows.at[b, pl.ds(j * _CH, _CH)], sem_g)
                    for j in range(_PCPI)]

        def scatters(b):
            return [pltpu.make_async_copy(
                        rows.at[b, pl.ds(j * _CH, _CH)],
                        acc.at[didx.at[b, j]], sem_s)
                    for j in range(_PCPI)]

        load_idx(0, 0)
        for cp in gathers(0):
            cp.start()

        def phase(p, b):
            for cp in gathers(b):
                cp.wait()
            for cp in scatters(b):
                cp.start(add=True)

            @pl.when(p + 1 < _PH)
            def _():
                @pl.when(p >= 1)
                def _():
                    for cp in scatters(1 - b):
                        cp.wait()
                load_idx(p + 1, 1 - b)
                for cp in gathers(1 - b):
                    cp.start()

        @pl.loop(0, _PH, step=2)
        def _(k):
            phase(k, 0)
            phase(k + 1, 1)

        for cp in scatters(0):
            cp.wait()
        for cp in scatters(1):
            cp.wait()
        plsc.subcore_barrier()
        pltpu.sync_copy(acc.at[pl.ds(r0, _ROWS)],
                        out_hbm.at[pl.ds(c * _NPAD + r0, _ROWS)])

    return edge_kernel(y, src, dst, zeros64)


def _dinv_block(deg_ref, n):
    degc = jnp.max(deg_ref[...], axis=1, keepdims=True)
    row = n * 128 + lax.broadcasted_iota(jnp.int32, (128, 1), 0)
    return jnp.where(row < _N, 1.0 / jnp.sqrt(degc + 1.0), 0.0)


def _k1_body(x_ref, w_ref, deg_ref, y_ref):
    dinv = _dinv_block(deg_ref, pl.program_id(0))
    xw = jnp.dot(x_ref[...], w_ref[...], preferred_element_type=jnp.float32,
                 precision=lax.Precision.HIGHEST)
    y_ref[...] = xw * dinv


def _tc_scale_matmul(x, w, deg):
    fin = w.shape[0]
    return pl.pallas_call(
        _k1_body,
        grid=(_NB,),
        in_specs=[
            pl.BlockSpec((128, fin), lambda n: (n, 0)),
            pl.BlockSpec((fin, 64), lambda n: (0, 0)),
            pl.BlockSpec((128, 16), lambda n: (n, 0)),
        ],
        out_specs=pl.BlockSpec((128, 64), lambda n: (n, 0)),
        out_shape=jax.ShapeDtypeStruct((_NPAD, 64), jnp.float32),
    )(x, w, deg)


def _k2_body(acc_ref, y_ref, deg_ref, b16_ref, gb_ref, miw_ref, mib_ref,
             h_ref, p_ref):
    n = pl.program_id(0)
    dinv = _dinv_block(deg_ref, n)
    h = jnp.maximum(
        dinv * (acc_ref[0] + acc_ref[1] + y_ref[...]) + gb_ref[...], 0.0)
    h_ref[...] = h
    d = jnp.maximum(
        jnp.dot(h, miw_ref[...], preferred_element_type=jnp.float32,
                precision=lax.Precision.HIGHEST)
        + mib_ref[...], 0.0)
    bc = jnp.max(b16_ref[...], axis=1, keepdims=True)
    oneh = (bc == lax.broadcasted_iota(jnp.int32, (128, _B), 1)
            .astype(jnp.float32))
    pp = lax.dot_general(oneh.astype(jnp.float32), d,
                         (((0,), (0,)), ((), ())),
                         preferred_element_type=jnp.float32,
                         precision=lax.Precision.HIGHEST)

    @pl.when(n == 0)
    def _():
        p_ref[...] = pp

    @pl.when(n != 0)
    def _():
        p_ref[...] = p_ref[...] + pp


def _tc_update_pool(acc2, y, deg, b16, gcn_b, mi_w, mi_b):
    return pl.pallas_call(
        _k2_body,
        grid=(_NB,),
        in_specs=[
            pl.BlockSpec((2, 128, 64), lambda n: (0, n, 0)),
            pl.BlockSpec((128, 64), lambda n: (n, 0)),
            pl.BlockSpec((128, 16), lambda n: (n, 0)),
            pl.BlockSpec((128, 16), lambda n: (n, 0)),
            pl.BlockSpec((1, 64), lambda n: (0, 0)),
            pl.BlockSpec((64, 64), lambda n: (0, 0)),
            pl.BlockSpec((1, 64), lambda n: (0, 0)),
        ],
        out_specs=[
            pl.BlockSpec((128, 64), lambda n: (n, 0)),
            pl.BlockSpec((_B, 64), lambda n: (0, 0)),
        ],
        out_shape=[
            jax.ShapeDtypeStruct((_NPAD, 64), jnp.float32),
            jax.ShapeDtypeStruct((_B, 64), jnp.float32),
        ],
    )(acc2, y, deg, b16, gcn_b, mi_w, mi_b)


def _k3_body(p10_ref, p20_ref, p11_ref, p21_ref, p12_ref, p22_ref,
             mow0_ref, mob0_ref, mow1_ref, mob1_ref, mow2_ref, mob2_ref,
             csw0a_ref, csw0b_ref, csw0c_ref, csb0_ref,
             csw1_ref, csb1_ref, scw0_ref, scb0_ref, scw1_ref, scb1_ref,
             out_ref):
    def diff(p1_ref, p2_ref, mow_ref, mob_ref):
        o1 = jnp.maximum(
            jnp.dot(p1_ref[...], mow_ref[...],
                    preferred_element_type=jnp.float32,
                    precision=lax.Precision.HIGHEST) + mob_ref[...], 0.0)
        o2 = jnp.maximum(
            jnp.dot(p2_ref[...], mow_ref[...],
                    preferred_element_type=jnp.float32,
                    precision=lax.Precision.HIGHEST) + mob_ref[...], 0.0)
        return jnp.exp(-jnp.square(o1 - o2))

    d0 = diff(p10_ref, p20_ref, mow0_ref, mob0_ref)
    d1 = diff(p11_ref, p21_ref, mow1_ref, mob1_ref)
    d2 = diff(p12_ref, p22_ref, mow2_ref, mob2_ref)
    h = (jnp.dot(d0, csw0a_ref[...], preferred_element_type=jnp.float32,
                 precision=lax.Precision.HIGHEST)
         + jnp.dot(d1, csw0b_ref[...], preferred_element_type=jnp.float32,
                   precision=lax.Precision.HIGHEST)
         + jnp.dot(d2, csw0c_ref[...], preferred_element_type=jnp.float32,
                   precision=lax.Precision.HIGHEST)
         + csb0_ref[...])
    h = jnp.maximum(h, 0.0)
    h = jnp.tanh(
        jnp.dot(h, csw1_ref[...], preferred_element_type=jnp.float32,
                precision=lax.Precision.HIGHEST)
        + csb1_ref[...])
    s = jnp.maximum(
        jnp.dot(h, scw0_ref[...], preferred_element_type=jnp.float32,
                precision=lax.Precision.HIGHEST)
        + scb0_ref[...], 0.0)
    z = (jnp.dot(s, scw1_ref[...], preferred_element_type=jnp.float32,
                 precision=lax.Precision.HIGHEST)
         + scb1_ref[...])
    out_ref[...] = 1.0 / (1.0 + jnp.exp(-z))


def _tc_head(pooled, mo, cs_w0, cs_b0, cs_w1, cs_b1,
             sc_w0, sc_b0, sc_w1, sc_b1):
    args = [pooled[0][0], pooled[0][1], pooled[1][0], pooled[1][1],
            pooled[2][0], pooled[2][1],
            mo[0][0], mo[0][1].reshape(1, -1),
            mo[1][0], mo[1][1].reshape(1, -1),
            mo[2][0], mo[2][1].reshape(1, -1),
            cs_w0[0:64], cs_w0[64:128], cs_w0[128:192],
            cs_b0.reshape(1, -1),
            cs_w1, cs_b1.reshape(1, -1),
            sc_w0, sc_b0.reshape(1, -1),
            sc_w1, sc_b1.reshape(1, 1)]
    return pl.pallas_call(
        _k3_body,
        out_shape=jax.ShapeDtypeStruct((_B, 1), jnp.float32),
    )(*args)


def kernel(features_1, features_2, edge_index_1, edge_index_2,
           batch_1, batch_2,
           gcn_W0, gcn_b0, mi_W0, mi_b0, mo_W0, mo_b0,
           gcn_W1, gcn_b1, mi_W1, mi_b1, mo_W1, mo_b1,
           gcn_W2, gcn_b2, mi_W2, mi_b2, mo_W2, mo_b2,
           cs_W0, cs_b0, cs_W1, cs_b1, sc_W0, sc_b0, sc_W1, sc_b1):
    f32 = jnp.float32
    epad = jnp.full((_EPAD - _E,), _N, jnp.int32)
    src1 = jnp.concatenate([edge_index_1[0], epad]).reshape(_ERB, _CH)
    dst1 = jnp.concatenate([edge_index_1[1], epad]).reshape(_ERB, _CH)
    src2 = jnp.concatenate([edge_index_2[0], epad]).reshape(_ERB, _CH)
    dst2 = jnp.concatenate([edge_index_2[1], epad]).reshape(_ERB, _CH)
    dstcat = jnp.concatenate([dst1, dst2], axis=0)

    zeros16 = jnp.zeros((_NPAD, 16), f32)
    zeros64 = jnp.zeros((_NPAD, 64), f32)
    ones16 = jnp.ones((_CH, 16), f32)

    degcat = _sc_degree(dstcat, ones16, zeros16).reshape(2, _NPAD, 16)
    deg = [degcat[0], degcat[1]]

    bpad = jnp.full((_NPAD - _N,), _B, jnp.int32)
    b16 = [jnp.broadcast_to(
               jnp.concatenate([b, bpad]).astype(f32)[:, None], (_NPAD, 16))
           for b in (batch_1, batch_2)]

    x = [jnp.pad(features_1, ((0, _NPAD - _N), (0, 0))),
         jnp.pad(features_2, ((0, _NPAD - _N), (0, 0)))]
    src = [src1, src2]
    dst = [dst1, dst2]

    gcn = [(gcn_W0, gcn_b0.reshape(1, -1)),
           (gcn_W1, gcn_b1.reshape(1, -1)),
           (gcn_W2, gcn_b2.reshape(1, -1))]
    mi = [(mi_W0, mi_b0.reshape(1, -1)),
          (mi_W1, mi_b1.reshape(1, -1)),
          (mi_W2, mi_b2.reshape(1, -1))]
    mo = [(mo_W0, mo_b0), (mo_W1, mo_b1), (mo_W2, mo_b2)]

    pooled = []
    for i in range(3):
        y, acc, p = [None, None], [None, None], [None, None]
        for g in range(2):
            y[g] = _tc_scale_matmul(x[g], gcn[i][0], deg[g])
            acc[g] = _sc_edge_scatter(y[g], src[g], dst[g], zeros64)
        for g in range(2):
            x[g], p[g] = _tc_update_pool(
                acc[g].reshape(2, _NPAD, 64), y[g], deg[g], b16[g],
                gcn[i][1], mi[i][0], mi[i][1])
        pooled.append(p)

    score = _tc_head(pooled, mo, cs_W0, cs_b0, cs_W1, cs_b1,
                     sc_W0, sc_b0, sc_W1, sc_b1)
    return score.reshape(-1)


# speedup vs baseline: 1.9494x; 1.9494x over previous
import functools

import jax
import jax.numpy as jnp
from jax import lax
from jax.experimental import pallas as pl
from jax.experimental.pallas import tpu as pltpu
from jax.experimental.pallas import tpu_sc as plsc

_N = 10000
_E = 320000
_B = 128
_NPAD = 10112
_CH = 128
_CPI = 8
_TILES = 16
_ITERS = 20
_CHUNKS = _CPI * _ITERS
_EPAD = _TILES * _CHUNKS * _CH
_ROWS = _NPAD // _TILES
_ERB = _EPAD // _CH
_NB = _NPAD // 128
_PCPI = 4
_WCH = _ERB // 32
_PH = _WCH // _PCPI


def _sc_degree(dst2, ones16, zeros16):
    mesh = plsc.VectorSubcoreMesh(core_axis_name="c", subcore_axis_name="s")

    @functools.partial(
        pl.kernel, mesh=mesh,
        compiler_params=pltpu.CompilerParams(use_tc_tiling_on_sc=False),
        out_type=jax.ShapeDtypeStruct((2 * _NPAD, 16), jnp.float32),
        scratch_types=[
            pltpu.VMEM((_CPI, _CH), jnp.int32),
            pltpu.VMEM((_CH, 16), jnp.float32),
            pltpu.VMEM_SHARED((_NPAD, 16), jnp.float32),
        ],
    )
    def deg_kernel(dst_hbm, ones_hbm, z_hbm, out_hbm, didx, ones_v, acc):
        c = lax.axis_index("c")
        s = lax.axis_index("s")
        r0 = s * _ROWS
        pltpu.sync_copy(z_hbm.at[pl.ds(r0, _ROWS)], acc.at[pl.ds(r0, _ROWS)])
        pltpu.sync_copy(ones_hbm, ones_v)
        plsc.subcore_barrier()
        rbase = c * _ERB + s * _CHUNKS

        def body(i, carry):
            pltpu.sync_copy(dst_hbm.at[pl.ds(rbase + i * _CPI, _CPI)], didx)
            for j in range(_CPI):
                pltpu.sync_copy(ones_v, acc.at[didx.at[j]], add=True)
            return carry

        lax.fori_loop(0, _ITERS, body, 0)
        plsc.subcore_barrier()
        pltpu.sync_copy(acc.at[pl.ds(r0, _ROWS)],
                        out_hbm.at[pl.ds(c * _NPAD + r0, _ROWS)])

    return deg_kernel(dst2, ones16, zeros16)


def _sc_edge_scatter(y, src, dst, zeros64):
    mesh = plsc.VectorSubcoreMesh(core_axis_name="c", subcore_axis_name="s")

    @functools.partial(
        pl.kernel, mesh=mesh,
        compiler_params=pltpu.CompilerParams(use_tc_tiling_on_sc=False),
        out_type=jax.ShapeDtypeStruct((2 * _NPAD, 64), jnp.float32),
        scratch_types=[
            pltpu.VMEM((2, _PCPI, _CH), jnp.int32),
            pltpu.VMEM((2, _PCPI, _CH), jnp.int32),
            pltpu.VMEM((2, _PCPI * _CH, 64), jnp.float32),
            pltpu.VMEM_SHARED((_NPAD, 64), jnp.float32),
            pltpu.SemaphoreType.DMA,
            pltpu.SemaphoreType.DMA,
        ],
    )
    def edge_kernel(y_hbm, src_hbm, dst_hbm, z_hbm, out_hbm,
                    sidx, didx, rows, acc, sem_g, sem_s):
        c = lax.axis_index("c")
        s = lax.axis_index("s")
        r0 = s * _ROWS
        pltpu.sync_copy(z_hbm.at[pl.ds(r0, _ROWS)], acc.at[pl.ds(r0, _ROWS)])
        plsc.subcore_barrier()
        rbase = (s * 2 + c) * _WCH

        def load_idx(p, b):
            off = rbase + p * _PCPI
            pltpu.sync_copy(src_hbm.at[pl.ds(off, _PCPI)], sidx.at[b])
            pltpu.sync_copy(dst_hbm.at[pl.ds(off, _PCPI)], didx.at[b])

        def gathers(b):
            return [pltpu.make_async_copy(
                        y_hbm.at[sidx.at[b, j]],
                        rows.at[b, pl.ds(j * _CH, _CH)], sem_g)
                    for j in range(_PCPI)]

        def scatters(b):
            return [pltpu.make_async_copy(
                        rows.at[b, pl.ds(j * _CH, _CH)],
                        acc.at[didx.at[b, j]], sem_s)
                    for j in range(_PCPI)]

        load_idx(0, 0)
        for cp in gathers(0):
            cp.start()

        def phase(p, b):
            for cp in gathers(b):
                cp.wait()
            for cp in scatters(b):
                cp.start(add=True)

            @pl.when(p + 1 < _PH)
            def _():
                @pl.when(p >= 1)
                def _():
                    for cp in scatters(1 - b):
                        cp.wait()
                load_idx(p + 1, 1 - b)
                for cp in gathers(1 - b):
                    cp.start()

        @pl.loop(0, _PH, step=2)
        def _(k):
            phase(k, 0)
            phase(k + 1, 1)

        for cp in scatters(0):
            cp.wait()
        for cp in scatters(1):
            cp.wait()
        plsc.subcore_barrier()
        pltpu.sync_copy(acc.at[pl.ds(r0, _ROWS)],
                        out_hbm.at[pl.ds(c * _NPAD + r0, _ROWS)])

    return edge_kernel(y, src, dst, zeros64)


def _dinv_block(deg_ref, n):
    degc = jnp.max(deg_ref[...], axis=1, keepdims=True)
    row = n * 128 + lax.broadcasted_iota(jnp.int32, (128, 1), 0)
    return jnp.where(row < _N, 1.0 / jnp.sqrt(degc + 1.0), 0.0)


def _k1_body(x_ref, w_ref, deg_ref, y_ref):
    dinv = _dinv_block(deg_ref, pl.program_id(0))
    xw = jnp.dot(x_ref[...], w_ref[...], preferred_element_type=jnp.float32,
                 precision=lax.Precision.HIGHEST)
    y_ref[...] = xw * dinv


def _tc_scale_matmul(x, w, deg):
    fin = w.shape[0]
    return pl.pallas_call(
        _k1_body,
        grid=(_NB,),
        in_specs=[
            pl.BlockSpec((128, fin), lambda n: (n, 0)),
            pl.BlockSpec((fin, 64), lambda n: (0, 0)),
            pl.BlockSpec((128, 16), lambda n: (n, 0)),
        ],
        out_specs=pl.BlockSpec((128, 64), lambda n: (n, 0)),
        out_shape=jax.ShapeDtypeStruct((_NPAD, 64), jnp.float32),
    )(x, w, deg)


def _k2_body(acc_ref, y_ref, deg_ref, b16_ref, gb_ref, miw_ref, mib_ref,
             h_ref, p_ref):
    n = pl.program_id(0)
    dinv = _dinv_block(deg_ref, n)
    h = jnp.maximum(
        dinv * (acc_ref[0] + acc_ref[1] + y_ref[...]) + gb_ref[...], 0.0)
    h_ref[...] = h
    d = jnp.maximum(
        jnp.dot(h, miw_ref[...], preferred_element_type=jnp.float32,
                precision=lax.Precision.HIGHEST)
        + mib_ref[...], 0.0)
    bc = jnp.max(b16_ref[...], axis=1, keepdims=True)
    oneh = (bc == lax.broadcasted_iota(jnp.int32, (128, _B), 1)
            .astype(jnp.float32))
    pp = lax.dot_general(oneh.astype(jnp.float32), d,
                         (((0,), (0,)), ((), ())),
                         preferred_element_type=jnp.float32,
                         precision=lax.Precision.HIGHEST)

    @pl.when(n == 0)
    def _():
        p_ref[...] = pp

    @pl.when(n != 0)
    def _():
        p_ref[...] = p_ref[...] + pp


def _tc_update_pool(acc2, y, deg, b16, gcn_b, mi_w, mi_b):
    return pl.pallas_call(
        _k2_body,
        grid=(_NB,),
        in_specs=[
            pl.BlockSpec((2, 128, 64), lambda n: (0, n, 0)),
            pl.BlockSpec((128, 64), lambda n: (n, 0)),
            pl.BlockSpec((128, 16), lambda n: (n, 0)),
            pl.BlockSpec((128, 16), lambda n: (n, 0)),
            pl.BlockSpec((1, 64), lambda n: (0, 0)),
            pl.BlockSpec((64, 64), lambda n: (0, 0)),
            pl.BlockSpec((1, 64), lambda n: (0, 0)),
        ],
        out_specs=[
            pl.BlockSpec((128, 64), lambda n: (n, 0)),
            pl.BlockSpec((_B, 64), lambda n: (0, 0)),
        ],
        out_shape=[
            jax.ShapeDtypeStruct((_NPAD, 64), jnp.float32),
            jax.ShapeDtypeStruct((_B, 64), jnp.float32),
        ],
    )(acc2, y, deg, b16, gcn_b, mi_w, mi_b)


def _k3_body(p10_ref, p20_ref, p11_ref, p21_ref, p12_ref, p22_ref,
             mow0_ref, mob0_ref, mow1_ref, mob1_ref, mow2_ref, mob2_ref,
             csw0a_ref, csw0b_ref, csw0c_ref, csb0_ref,
             csw1_ref, csb1_ref, scw0_ref, scb0_ref, scw1_ref, scb1_ref,
             out_ref):
    def diff(p1_ref, p2_ref, mow_ref, mob_ref):
        o1 = jnp.maximum(
            jnp.dot(p1_ref[...], mow_ref[...],
                    preferred_element_type=jnp.float32,
                    precision=lax.Precision.HIGHEST) + mob_ref[...], 0.0)
        o2 = jnp.maximum(
            jnp.dot(p2_ref[...], mow_ref[...],
                    preferred_element_type=jnp.float32,
                    precision=lax.Precision.HIGHEST) + mob_ref[...], 0.0)
        return jnp.exp(-jnp.square(o1 - o2))

    d0 = diff(p10_ref, p20_ref, mow0_ref, mob0_ref)
    d1 = diff(p11_ref, p21_ref, mow1_ref, mob1_ref)
    d2 = diff(p12_ref, p22_ref, mow2_ref, mob2_ref)
    h = (jnp.dot(d0, csw0a_ref[...], preferred_element_type=jnp.float32,
                 precision=lax.Precision.HIGHEST)
         + jnp.dot(d1, csw0b_ref[...], preferred_element_type=jnp.float32,
                   precision=lax.Precision.HIGHEST)
         + jnp.dot(d2, csw0c_ref[...], preferred_element_type=jnp.float32,
                   precision=lax.Precision.HIGHEST)
         + csb0_ref[...])
    h = jnp.maximum(h, 0.0)
    h = jnp.tanh(
        jnp.dot(h, csw1_ref[...], preferred_element_type=jnp.float32,
                precision=lax.Precision.HIGHEST)
        + csb1_ref[...])
    s = jnp.maximum(
        jnp.dot(h, scw0_ref[...], preferred_element_type=jnp.float32,
                precision=lax.Precision.HIGHEST)
        + scb0_ref[...], 0.0)
    z = (jnp.dot(s, scw1_ref[...], preferred_element_type=jnp.float32,
                 precision=lax.Precision.HIGHEST)
         + scb1_ref[...])
    out_ref[...] = 1.0 / (1.0 + jnp.exp(-z))


def _tc_head(pooled, mo, cs_w0, cs_b0, cs_w1, cs_b1,
             sc_w0, sc_b0, sc_w1, sc_b1):
    args = [pooled[0][0], pooled[0][1], pooled[1][0], pooled[1][1],
            pooled[2][0], pooled[2][1],
            mo[0][0], mo[0][1].reshape(1, -1),
            mo[1][0], mo[1][1].reshape(1, -1),
            mo[2][0], mo[2][1].reshape(1, -1),
            cs_w0[0:64], cs_w0[64:128], cs_w0[128:192],
            cs_b0.reshape(1, -1),
            cs_w1, cs_b1.reshape(1, -1),
            sc_w0, sc_b0.reshape(1, -1),
            sc_w1, sc_b1.reshape(1, 1)]
    return pl.pallas_call(
        _k3_body,
        out_shape=jax.ShapeDtypeStruct((_B, 1), jnp.float32),
    )(*args)


def kernel(features_1, features_2, edge_index_1, edge_index_2,
           batch_1, batch_2,
           gcn_W0, gcn_b0, mi_W0, mi_b0, mo_W0, mo_b0,
           gcn_W1, gcn_b1, mi_W1, mi_b1, mo_W1, mo_b1,
           gcn_W2, gcn_b2, mi_W2, mi_b2, mo_W2, mo_b2,
           cs_W0, cs_b0, cs_W1, cs_b1, sc_W0, sc_b0, sc_W1, sc_b1):
    f32 = jnp.float32
    epad = _N + (jnp.arange(_EPAD - _E, dtype=jnp.int32) % (_NPAD - _N))
    src1 = jnp.concatenate([edge_index_1[0], epad]).reshape(_ERB, _CH)
    dst1 = jnp.concatenate([edge_index_1[1], epad]).reshape(_ERB, _CH)
    src2 = jnp.concatenate([edge_index_2[0], epad]).reshape(_ERB, _CH)
    dst2 = jnp.concatenate([edge_index_2[1], epad]).reshape(_ERB, _CH)
    dstcat = jnp.concatenate([dst1, dst2], axis=0)

    zeros16 = jnp.zeros((_NPAD, 16), f32)
    zeros64 = jnp.zeros((_NPAD, 64), f32)
    ones16 = jnp.ones((_CH, 16), f32)

    degcat = _sc_degree(dstcat, ones16, zeros16).reshape(2, _NPAD, 16)
    deg = [degcat[0], degcat[1]]

    bpad = jnp.full((_NPAD - _N,), _B, jnp.int32)
    b16 = [jnp.broadcast_to(
               jnp.concatenate([b, bpad]).astype(f32)[:, None], (_NPAD, 16))
           for b in (batch_1, batch_2)]

    x = [jnp.pad(features_1, ((0, _NPAD - _N), (0, 0))),
         jnp.pad(features_2, ((0, _NPAD - _N), (0, 0)))]
    src = [src1, src2]
    dst = [dst1, dst2]

    gcn = [(gcn_W0, gcn_b0.reshape(1, -1)),
           (gcn_W1, gcn_b1.reshape(1, -1)),
           (gcn_W2, gcn_b2.reshape(1, -1))]
    mi = [(mi_W0, mi_b0.reshape(1, -1)),
          (mi_W1, mi_b1.reshape(1, -1)),
          (mi_W2, mi_b2.reshape(1, -1))]
    mo = [(mo_W0, mo_b0), (mo_W1, mo_b1), (mo_W2, mo_b2)]

    pooled = []
    for i in range(3):
        y, acc, p = [None, None], [None, None], [None, None]
        for g in range(2):
            y[g] = _tc_scale_matmul(x[g], gcn[i][0], deg[g])
            acc[g] = _sc_edge_scatter(y[g], src[g], dst[g], zeros64)
        for g in range(2):
            x[g], p[g] = _tc_update_pool(
                acc[g].reshape(2, _NPAD, 64), y[g], deg[g], b16[g],
                gcn[i][1], mi[i][0], mi[i][1])
        pooled.append(p)

    score = _tc_head(pooled, mo, cs_W0, cs_b0, cs_W1, cs_b1,
                     sc_W0, sc_b0, sc_W1, sc_b1)
    return score.reshape(-1)


# speedup vs baseline: 2.9387x; 1.5075x over previous
import functools

import jax
import jax.numpy as jnp
from jax import lax
from jax.experimental import pallas as pl
from jax.experimental.pallas import tpu as pltpu
from jax.experimental.pallas import tpu_sc as plsc

_N = 10000
_E = 320000
_B = 128
_NPAD = 10240
_CH = 128
_CPI = 8
_TILES = 16
_ITERS = 20
_CHUNKS = _CPI * _ITERS
_EPAD = _TILES * _CHUNKS * _CH
_ROWS = _NPAD // _TILES
_ERB = _EPAD // _CH
_BLK = 512
_NBLK = _NPAD // _BLK
_PCPI = 4
_WCH = _ERB // 32
_PH = _WCH // _PCPI


def _sc_degree(dst2, ones16, zeros16):
    mesh = plsc.VectorSubcoreMesh(core_axis_name="c", subcore_axis_name="s")

    @functools.partial(
        pl.kernel, mesh=mesh,
        compiler_params=pltpu.CompilerParams(use_tc_tiling_on_sc=False),
        out_type=jax.ShapeDtypeStruct((2 * _NPAD, 16), jnp.float32),
        scratch_types=[
            pltpu.VMEM((_CPI, _CH), jnp.int32),
            pltpu.VMEM((_CH, 16), jnp.float32),
            pltpu.VMEM_SHARED((_NPAD, 16), jnp.float32),
            pltpu.SemaphoreType.DMA,
        ],
    )
    def deg_kernel(dst_hbm, ones_hbm, z_hbm, out_hbm, didx, ones_v, acc, sem):
        c = lax.axis_index("c")
        s = lax.axis_index("s")
        r0 = s * _ROWS
        pltpu.sync_copy(z_hbm.at[pl.ds(r0, _ROWS)], acc.at[pl.ds(r0, _ROWS)])
        pltpu.sync_copy(ones_hbm, ones_v)
        plsc.subcore_barrier()
        rbase = c * _ERB + s * _CHUNKS

        def body(i, carry):
            pltpu.sync_copy(dst_hbm.at[pl.ds(rbase + i * _CPI, _CPI)], didx)
            cps = [pltpu.make_async_copy(ones_v, acc.at[didx.at[j]], sem)
                   for j in range(_CPI)]
            for cp in cps:
                cp.start(add=True)
            for cp in cps:
                cp.wait()
            return carry

        lax.fori_loop(0, _ITERS, body, 0)
        plsc.subcore_barrier()
        pltpu.sync_copy(acc.at[pl.ds(r0, _ROWS)],
                        out_hbm.at[pl.ds(c * _NPAD + r0, _ROWS)])

    return deg_kernel(dst2, ones16, zeros16)


def _sc_edge_scatter(y, src, dst, zeros64):
    mesh = plsc.VectorSubcoreMesh(core_axis_name="c", subcore_axis_name="s")

    @functools.partial(
        pl.kernel, mesh=mesh,
        compiler_params=pltpu.CompilerParams(use_tc_tiling_on_sc=False),
        out_type=jax.ShapeDtypeStruct((2 * _NPAD, 64), jnp.float32),
        scratch_types=[
            pltpu.VMEM((2, _PCPI, _CH), jnp.int32),
            pltpu.VMEM((2, _PCPI, _CH), jnp.int32),
            pltpu.VMEM((2, _PCPI * _CH, 64), jnp.float32),
            pltpu.VMEM_SHARED((_NPAD, 64), jnp.float32),
            pltpu.SemaphoreType.DMA,
            pltpu.SemaphoreType.DMA,
        ],
    )
    def edge_kernel(y_hbm, src_hbm, dst_hbm, z_hbm, out_hbm,
                    sidx, didx, rows, acc, sem_g, sem_s):
        c = lax.axis_index("c")
        s = lax.axis_index("s")
        r0 = s * _ROWS
        pltpu.sync_copy(z_hbm.at[pl.ds(r0, _ROWS)], acc.at[pl.ds(r0, _ROWS)])
        plsc.subcore_barrier()
        rbase = (s * 2 + c) * _WCH

        def load_idx(p, b):
            off = rbase + p * _PCPI
            pltpu.sync_copy(src_hbm.at[pl.ds(off, _PCPI)], sidx.at[b])
            pltpu.sync_copy(dst_hbm.at[pl.ds(off, _PCPI)], didx.at[b])

        def gathers(b):
            return [pltpu.make_async_copy(
                        y_hbm.at[sidx.at[b, j]],
                        rows.at[b, pl.ds(j * _CH, _CH)], sem_g)
                    for j in range(_PCPI)]

        def scatters(b):
            return [pltpu.make_async_copy(
                        rows.at[b, pl.ds(j * _CH, _CH)],
                        acc.at[didx.at[b, j]], sem_s)
                    for j in range(_PCPI)]

        load_idx(0, 0)
        for cp in gathers(0):
            cp.start()

        def phase(p, b):
            for cp in gathers(b):
                cp.wait()
            for cp in scatters(b):
                cp.start(add=True)

            @pl.when(p + 1 < _PH)
            def _():
                @pl.when(p >= 1)
                def _():
                    for cp in scatters(1 - b):
                        cp.wait()
                load_idx(p + 1, 1 - b)
                for cp in gathers(1 - b):
                    cp.start()

        @pl.loop(0, _PH, step=2)
        def _(k):
            phase(k, 0)
            phase(k + 1, 1)

        for cp in scatters(0):
            cp.wait()
        for cp in scatters(1):
            cp.wait()
        plsc.subcore_barrier()
        pltpu.sync_copy(acc.at[pl.ds(r0, _ROWS)],
                        out_hbm.at[pl.ds(c * _NPAD + r0, _ROWS)])

    return edge_kernel(y, src, dst, zeros64)


def _dinv_block(deg_ref, n):
    degc = jnp.max(deg_ref[...], axis=1, keepdims=True)
    row = n * _BLK + lax.broadcasted_iota(jnp.int32, (_BLK, 1), 0)
    return jnp.where(row < _N, 1.0 / jnp.sqrt(degc + 1.0), 0.0)


def _k1_body(x_ref, w_ref, deg_ref, y_ref):
    dinv = _dinv_block(deg_ref, pl.program_id(0))
    xw = jnp.dot(x_ref[...], w_ref[...], preferred_element_type=jnp.float32)
    y_ref[...] = xw * dinv


def _tc_scale_matmul(x, w, deg):
    fin = w.shape[0]
    return pl.pallas_call(
        _k1_body,
        grid=(_NBLK,),
        in_specs=[
            pl.BlockSpec((_BLK, fin), lambda n: (n, 0)),
            pl.BlockSpec((fin, 64), lambda n: (0, 0)),
            pl.BlockSpec((_BLK, 16), lambda n: (n, 0)),
        ],
        out_specs=pl.BlockSpec((_BLK, 64), lambda n: (n, 0)),
        out_shape=jax.ShapeDtypeStruct((_NPAD, 64), jnp.float32),
    )(x, w, deg)


def _layer_block(n, acc_ref, y_ref, deg_ref, b16_ref, gb_ref, miw_ref,
                 mib_ref):
    dinv = _dinv_block(deg_ref, n)
    h = jnp.maximum(
        dinv * (acc_ref[0] + acc_ref[1] + y_ref[...]) + gb_ref[...], 0.0)
    d = jnp.maximum(
        jnp.dot(h, miw_ref[...], preferred_element_type=jnp.float32)
        + mib_ref[...], 0.0)
    bc = jnp.max(b16_ref[...], axis=1, keepdims=True)
    oneh = (bc == lax.broadcasted_iota(jnp.int32, (_BLK, _B), 1)
            .astype(jnp.float32))
    pp = lax.dot_general(oneh.astype(jnp.float32), d,
                         (((0,), (0,)), ((), ())),
                         preferred_element_type=jnp.float32)
    return h, dinv, pp


def _k21_body(acc_ref, y_ref, deg_ref, b16_ref, gb_ref, miw_ref, mib_ref,
              wn_ref, yn_ref, p_ref):
    n = pl.program_id(0)
    h, dinv, pp = _layer_block(n, acc_ref, y_ref, deg_ref, b16_ref, gb_ref,
                               miw_ref, mib_ref)
    yn_ref[...] = jnp.dot(
        h, wn_ref[...], preferred_element_type=jnp.float32) * dinv

    @pl.when(n == 0)
    def _():
        p_ref[...] = pp

    @pl.when(n != 0)
    def _():
        p_ref[...] = p_ref[...] + pp


def _tc_update_pool_next(acc2, y, deg, b16, gcn_b, mi_w, mi_b, w_next):
    return pl.pallas_call(
        _k21_body,
        grid=(_NBLK,),
        in_specs=[
            pl.BlockSpec((2, _BLK, 64), lambda n: (0, n, 0)),
            pl.BlockSpec((_BLK, 64), lambda n: (n, 0)),
            pl.BlockSpec((_BLK, 16), lambda n: (n, 0)),
            pl.BlockSpec((_BLK, 16), lambda n: (n, 0)),
            pl.BlockSpec((1, 64), lambda n: (0, 0)),
            pl.BlockSpec((64, 64), lambda n: (0, 0)),
            pl.BlockSpec((1, 64), lambda n: (0, 0)),
            pl.BlockSpec((64, 64), lambda n: (0, 0)),
        ],
        out_specs=[
            pl.BlockSpec((_BLK, 64), lambda n: (n, 0)),
            pl.BlockSpec((_B, 64), lambda n: (0, 0)),
        ],
        out_shape=[
            jax.ShapeDtypeStruct((_NPAD, 64), jnp.float32),
            jax.ShapeDtypeStruct((_B, 64), jnp.float32),
        ],
    )(acc2, y, deg, b16, gcn_b, mi_w, mi_b, w_next)


def _k2f_body(acc_ref, y_ref, deg_ref, b16_ref, gb_ref, miw_ref, mib_ref,
              p_ref):
    n = pl.program_id(0)
    _, _, pp = _layer_block(n, acc_ref, y_ref, deg_ref, b16_ref, gb_ref,
                            miw_ref, mib_ref)

    @pl.when(n == 0)
    def _():
        p_ref[...] = pp

    @pl.when(n != 0)
    def _():
        p_ref[...] = p_ref[...] + pp


def _tc_update_pool_final(acc2, y, deg, b16, gcn_b, mi_w, mi_b):
    return pl.pallas_call(
        _k2f_body,
        grid=(_NBLK,),
        in_specs=[
            pl.BlockSpec((2, _BLK, 64), lambda n: (0, n, 0)),
            pl.BlockSpec((_BLK, 64), lambda n: (n, 0)),
            pl.BlockSpec((_BLK, 16), lambda n: (n, 0)),
            pl.BlockSpec((_BLK, 16), lambda n: (n, 0)),
            pl.BlockSpec((1, 64), lambda n: (0, 0)),
            pl.BlockSpec((64, 64), lambda n: (0, 0)),
            pl.BlockSpec((1, 64), lambda n: (0, 0)),
        ],
        out_specs=pl.BlockSpec((_B, 64), lambda n: (0, 0)),
        out_shape=jax.ShapeDtypeStruct((_B, 64), jnp.float32),
    )(acc2, y, deg, b16, gcn_b, mi_w, mi_b)


def _k3_body(p10_ref, p20_ref, p11_ref, p21_ref, p12_ref, p22_ref,
             mow0_ref, mob0_ref, mow1_ref, mob1_ref, mow2_ref, mob2_ref,
             csw0a_ref, csw0b_ref, csw0c_ref, csb0_ref,
             csw1_ref, csb1_ref, scw0_ref, scb0_ref, scw1_ref, scb1_ref,
             out_ref):
    def diff(p1_ref, p2_ref, mow_ref, mob_ref):
        o1 = jnp.maximum(
            jnp.dot(p1_ref[...], mow_ref[...],
                    preferred_element_type=jnp.float32) + mob_ref[...], 0.0)
        o2 = jnp.maximum(
            jnp.dot(p2_ref[...], mow_ref[...],
                    preferred_element_type=jnp.float32) + mob_ref[...], 0.0)
        return jnp.exp(-jnp.square(o1 - o2))

    d0 = diff(p10_ref, p20_ref, mow0_ref, mob0_ref)
    d1 = diff(p11_ref, p21_ref, mow1_ref, mob1_ref)
    d2 = diff(p12_ref, p22_ref, mow2_ref, mob2_ref)
    h = (jnp.dot(d0, csw0a_ref[...], preferred_element_type=jnp.float32)
         + jnp.dot(d1, csw0b_ref[...], preferred_element_type=jnp.float32)
         + jnp.dot(d2, csw0c_ref[...], preferred_element_type=jnp.float32)
         + csb0_ref[...])
    h = jnp.maximum(h, 0.0)
    h = jnp.tanh(
        jnp.dot(h, csw1_ref[...], preferred_element_type=jnp.float32)
        + csb1_ref[...])
    s = jnp.maximum(
        jnp.dot(h, scw0_ref[...], preferred_element_type=jnp.float32)
        + scb0_ref[...], 0.0)
    z = (jnp.dot(s, scw1_ref[...], preferred_element_type=jnp.float32)
         + scb1_ref[...])
    out_ref[...] = 1.0 / (1.0 + jnp.exp(-z))


def _tc_head(pooled, mo, cs_w0, cs_b0, cs_w1, cs_b1,
             sc_w0, sc_b0, sc_w1, sc_b1):
    args = [pooled[0][0], pooled[0][1], pooled[1][0], pooled[1][1],
            pooled[2][0], pooled[2][1],
            mo[0][0], mo[0][1].reshape(1, -1),
            mo[1][0], mo[1][1].reshape(1, -1),
            mo[2][0], mo[2][1].reshape(1, -1),
            cs_w0[0:64], cs_w0[64:128], cs_w0[128:192],
            cs_b0.reshape(1, -1),
            cs_w1, cs_b1.reshape(1, -1),
            sc_w0, sc_b0.reshape(1, -1),
            sc_w1, sc_b1.reshape(1, 1)]
    return pl.pallas_call(
        _k3_body,
        out_shape=jax.ShapeDtypeStruct((_B, 1), jnp.float32),
    )(*args)


def kernel(features_1, features_2, edge_index_1, edge_index_2,
           batch_1, batch_2,
           gcn_W0, gcn_b0, mi_W0, mi_b0, mo_W0, mo_b0,
           gcn_W1, gcn_b1, mi_W1, mi_b1, mo_W1, mo_b1,
           gcn_W2, gcn_b2, mi_W2, mi_b2, mo_W2, mo_b2,
           cs_W0, cs_b0, cs_W1, cs_b1, sc_W0, sc_b0, sc_W1, sc_b1):
    f32 = jnp.float32
    epad = _N + (jnp.arange(_EPAD - _E, dtype=jnp.int32) % (_NPAD - _N))
    src1 = jnp.concatenate([edge_index_1[0], epad]).reshape(_ERB, _CH)
    dst1 = jnp.concatenate([edge_index_1[1], epad]).reshape(_ERB, _CH)
    src2 = jnp.concatenate([edge_index_2[0], epad]).reshape(_ERB, _CH)
    dst2 = jnp.concatenate([edge_index_2[1], epad]).reshape(_ERB, _CH)
    dstcat = jnp.concatenate([dst1, dst2], axis=0)

    zeros16 = jnp.zeros((_NPAD, 16), f32)
    zeros64 = jnp.zeros((_NPAD, 64), f32)
    ones16 = jnp.ones((_CH, 16), f32)

    degcat = _sc_degree(dstcat, ones16, zeros16).reshape(2, _NPAD, 16)
    deg = [degcat[0], degcat[1]]

    bpad = jnp.full((_NPAD - _N,), _B, jnp.int32)
    b16 = [jnp.broadcast_to(
               jnp.concatenate([b, bpad]).astype(f32)[:, None], (_NPAD, 16))
           for b in (batch_1, batch_2)]

    x = [jnp.pad(features_1, ((0, _NPAD - _N), (0, 0))),
         jnp.pad(features_2, ((0, _NPAD - _N), (0, 0)))]
    src = [src1, src2]
    dst = [dst1, dst2]

    gcn = [(gcn_W0, gcn_b0.reshape(1, -1)),
           (gcn_W1, gcn_b1.reshape(1, -1)),
           (gcn_W2, gcn_b2.reshape(1, -1))]
    mi = [(mi_W0, mi_b0.reshape(1, -1)),
          (mi_W1, mi_b1.reshape(1, -1)),
          (mi_W2, mi_b2.reshape(1, -1))]
    mo = [(mo_W0, mo_b0), (mo_W1, mo_b1), (mo_W2, mo_b2)]

    y = [_tc_scale_matmul(x[0], gcn[0][0], deg[0]),
         _tc_scale_matmul(x[1], gcn[0][0], deg[1])]
    pooled = []
    for i in range(3):
        acc, p = [None, None], [None, None]
        for g in range(2):
            acc[g] = _sc_edge_scatter(y[g], src[g], dst[g], zeros64)
        if i < 2:
            for g in range(2):
                y[g], p[g] = _tc_update_pool_next(
                    acc[g].reshape(2, _NPAD, 64), y[g], deg[g], b16[g],
                    gcn[i][1], mi[i][0], mi[i][1], gcn[i + 1][0])
        else:
            for g in range(2):
                p[g] = _tc_update_pool_final(
                    acc[g].reshape(2, _NPAD, 64), y[g], deg[g], b16[g],
                    gcn[i][1], mi[i][0], mi[i][1])
        pooled.append(p)

    score = _tc_head(pooled, mo, cs_W0, cs_b0, cs_W1, cs_b1,
                     sc_W0, sc_b0, sc_W1, sc_b1)
    return score.reshape(-1)


# speedup vs baseline: 3.0892x; 1.0512x over previous
import functools

import jax
import jax.numpy as jnp
from jax import lax
from jax.experimental import pallas as pl
from jax.experimental.pallas import tpu as pltpu
from jax.experimental.pallas import tpu_sc as plsc

_N = 10000
_E = 320000
_B = 128
_NPAD = 10240
_CH = 128
_CPI = 8
_TILES = 16
_ITERS = 20
_CHUNKS = _CPI * _ITERS
_EPAD = _TILES * _CHUNKS * _CH
_ROWS = _NPAD // _TILES
_ERB = _EPAD // _CH
_BLK = 512
_NBLK = _NPAD // _BLK
_PCPI = 5
_WCH = _ERB // 32
_PH = _WCH // _PCPI


def _sc_degree(dst2, ones16, zeros16):
    mesh = plsc.VectorSubcoreMesh(core_axis_name="c", subcore_axis_name="s")

    @functools.partial(
        pl.kernel, mesh=mesh,
        compiler_params=pltpu.CompilerParams(use_tc_tiling_on_sc=False),
        out_type=jax.ShapeDtypeStruct((2 * _NPAD, 16), jnp.float32),
        scratch_types=[
            pltpu.VMEM((_CPI, _CH), jnp.int32),
            pltpu.VMEM((_CH, 16), jnp.float32),
            pltpu.VMEM_SHARED((_NPAD, 16), jnp.float32),
            pltpu.SemaphoreType.DMA,
        ],
    )
    def deg_kernel(dst_hbm, ones_hbm, z_hbm, out_hbm, didx, ones_v, acc, sem):
        c = lax.axis_index("c")
        s = lax.axis_index("s")
        r0 = s * _ROWS
        pltpu.sync_copy(z_hbm.at[pl.ds(r0, _ROWS)], acc.at[pl.ds(r0, _ROWS)])
        pltpu.sync_copy(ones_hbm, ones_v)
        plsc.subcore_barrier()
        rbase = c * _ERB + s * _CHUNKS

        def body(i, carry):
            pltpu.sync_copy(dst_hbm.at[pl.ds(rbase + i * _CPI, _CPI)], didx)
            cps = [pltpu.make_async_copy(ones_v, acc.at[didx.at[j]], sem)
                   for j in range(_CPI)]
            for cp in cps:
                cp.start(add=True)
            for cp in cps:
                cp.wait()
            return carry

        lax.fori_loop(0, _ITERS, body, 0)
        plsc.subcore_barrier()
        pltpu.sync_copy(acc.at[pl.ds(r0, _ROWS)],
                        out_hbm.at[pl.ds(c * _NPAD + r0, _ROWS)])

    return deg_kernel(dst2, ones16, zeros16)


def _sc_edge_scatter(y, src, dst, zeros64):
    mesh = plsc.VectorSubcoreMesh(core_axis_name="c", subcore_axis_name="s")

    @functools.partial(
        pl.kernel, mesh=mesh,
        compiler_params=pltpu.CompilerParams(use_tc_tiling_on_sc=False),
        out_type=jax.ShapeDtypeStruct((2 * _NPAD, 64), jnp.float32),
        scratch_types=[
            pltpu.VMEM((2, _PCPI, _CH), jnp.int32),
            pltpu.VMEM((2, _PCPI, _CH), jnp.int32),
            pltpu.VMEM((2, _PCPI * _CH, 64), jnp.float32),
            pltpu.VMEM_SHARED((_NPAD, 64), jnp.float32),
            pltpu.SemaphoreType.DMA,
            pltpu.SemaphoreType.DMA,
        ],
    )
    def edge_kernel(y_hbm, src_hbm, dst_hbm, z_hbm, out_hbm,
                    sidx, didx, rows, acc, sem_g, sem_s):
        c = lax.axis_index("c")
        s = lax.axis_index("s")
        r0 = s * _ROWS
        pltpu.sync_copy(z_hbm.at[pl.ds(r0, _ROWS)], acc.at[pl.ds(r0, _ROWS)])
        plsc.subcore_barrier()
        rbase = (s * 2 + c) * _WCH

        def load_idx(p, b):
            off = rbase + p * _PCPI
            pltpu.sync_copy(src_hbm.at[pl.ds(off, _PCPI)], sidx.at[b])
            pltpu.sync_copy(dst_hbm.at[pl.ds(off, _PCPI)], didx.at[b])

        def gathers(b):
            return [pltpu.make_async_copy(
                        y_hbm.at[sidx.at[b, j]],
                        rows.at[b, pl.ds(j * _CH, _CH)], sem_g)
                    for j in range(_PCPI)]

        def scatters(b):
            return [pltpu.make_async_copy(
                        rows.at[b, pl.ds(j * _CH, _CH)],
                        acc.at[didx.at[b, j]], sem_s)
                    for j in range(_PCPI)]

        load_idx(0, 0)
        for cp in gathers(0):
            cp.start()

        def phase(p, b):
            for cp_g, cp_s in zip(gathers(b), scatters(b)):
                cp_g.wait()
                cp_s.start(add=True)

            @pl.when(p + 1 < _PH)
            def _():
                @pl.when(p >= 1)
                def _():
                    for cp in scatters(1 - b):
                        cp.wait()
                load_idx(p + 1, 1 - b)
                for cp in gathers(1 - b):
                    cp.start()

        @pl.loop(0, _PH, step=2)
        def _(k):
            phase(k, 0)
            phase(k + 1, 1)

        for cp in scatters(0):
            cp.wait()
        for cp in scatters(1):
            cp.wait()
        plsc.subcore_barrier()
        pltpu.sync_copy(acc.at[pl.ds(r0, _ROWS)],
                        out_hbm.at[pl.ds(c * _NPAD + r0, _ROWS)])

    return edge_kernel(y, src, dst, zeros64)


def _dinv_block(deg_ref, n):
    degc = jnp.max(deg_ref[...], axis=1, keepdims=True)
    row = n * _BLK + lax.broadcasted_iota(jnp.int32, (_BLK, 1), 0)
    return jnp.where(row < _N, 1.0 / jnp.sqrt(degc + 1.0), 0.0)


def _k1_body(x_ref, w_ref, deg_ref, y_ref):
    dinv = _dinv_block(deg_ref, pl.program_id(0))
    xw = jnp.dot(x_ref[...], w_ref[...], preferred_element_type=jnp.float32)
    y_ref[...] = xw * dinv


def _tc_scale_matmul(x, w, deg):
    fin = w.shape[0]
    return pl.pallas_call(
        _k1_body,
        grid=(_NBLK,),
        in_specs=[
            pl.BlockSpec((_BLK, fin), lambda n: (n, 0)),
            pl.BlockSpec((fin, 64), lambda n: (0, 0)),
            pl.BlockSpec((_BLK, 16), lambda n: (n, 0)),
        ],
        out_specs=pl.BlockSpec((_BLK, 64), lambda n: (n, 0)),
        out_shape=jax.ShapeDtypeStruct((_NPAD, 64), jnp.float32),
    )(x, w, deg)


def _layer_block(n, acc_ref, y_ref, deg_ref, b16_ref, gb_ref, miw_ref,
                 mib_ref):
    dinv = _dinv_block(deg_ref, n)
    h = jnp.maximum(
        dinv * (acc_ref[0] + acc_ref[1] + y_ref[...]) + gb_ref[...], 0.0)
    d = jnp.maximum(
        jnp.dot(h, miw_ref[...], preferred_element_type=jnp.float32)
        + mib_ref[...], 0.0)
    bc = jnp.max(b16_ref[...], axis=1, keepdims=True)
    oneh = (bc == lax.broadcasted_iota(jnp.int32, (_BLK, _B), 1)
            .astype(jnp.float32))
    pp = lax.dot_general(oneh.astype(jnp.float32), d,
                         (((0,), (0,)), ((), ())),
                         preferred_element_type=jnp.float32)
    return h, dinv, pp


def _k21_body(acc_ref, y_ref, deg_ref, b16_ref, gb_ref, miw_ref, mib_ref,
              wn_ref, yn_ref, p_ref):
    n = pl.program_id(0)
    h, dinv, pp = _layer_block(n, acc_ref, y_ref, deg_ref, b16_ref, gb_ref,
                               miw_ref, mib_ref)
    yn_ref[...] = jnp.dot(
        h, wn_ref[...], preferred_element_type=jnp.float32) * dinv

    @pl.when(n == 0)
    def _():
        p_ref[...] = pp

    @pl.when(n != 0)
    def _():
        p_ref[...] = p_ref[...] + pp


def _tc_update_pool_next(acc2, y, deg, b16, gcn_b, mi_w, mi_b, w_next):
    return pl.pallas_call(
        _k21_body,
        grid=(_NBLK,),
        in_specs=[
            pl.BlockSpec((2, _BLK, 64), lambda n: (0, n, 0)),
            pl.BlockSpec((_BLK, 64), lambda n: (n, 0)),
            pl.BlockSpec((_BLK, 16), lambda n: (n, 0)),
            pl.BlockSpec((_BLK, 16), lambda n: (n, 0)),
            pl.BlockSpec((1, 64), lambda n: (0, 0)),
            pl.BlockSpec((64, 64), lambda n: (0, 0)),
            pl.BlockSpec((1, 64), lambda n: (0, 0)),
            pl.BlockSpec((64, 64), lambda n: (0, 0)),
        ],
        out_specs=[
            pl.BlockSpec((_BLK, 64), lambda n: (n, 0)),
            pl.BlockSpec((_B, 64), lambda n: (0, 0)),
        ],
        out_shape=[
            jax.ShapeDtypeStruct((_NPAD, 64), jnp.float32),
            jax.ShapeDtypeStruct((_B, 64), jnp.float32),
        ],
    )(acc2, y, deg, b16, gcn_b, mi_w, mi_b, w_next)


def _k2f_body(acc_ref, y_ref, deg_ref, b16_ref, gb_ref, miw_ref, mib_ref,
              p_ref):
    n = pl.program_id(0)
    _, _, pp = _layer_block(n, acc_ref, y_ref, deg_ref, b16_ref, gb_ref,
                            miw_ref, mib_ref)

    @pl.when(n == 0)
    def _():
        p_ref[...] = pp

    @pl.when(n != 0)
    def _():
        p_ref[...] = p_ref[...] + pp


def _tc_update_pool_final(acc2, y, deg, b16, gcn_b, mi_w, mi_b):
    return pl.pallas_call(
        _k2f_body,
        grid=(_NBLK,),
        in_specs=[
            pl.BlockSpec((2, _BLK, 64), lambda n: (0, n, 0)),
            pl.BlockSpec((_BLK, 64), lambda n: (n, 0)),
            pl.BlockSpec((_BLK, 16), lambda n: (n, 0)),
            pl.BlockSpec((_BLK, 16), lambda n: (n, 0)),
            pl.BlockSpec((1, 64), lambda n: (0, 0)),
            pl.BlockSpec((64, 64), lambda n: (0, 0)),
            pl.BlockSpec((1, 64), lambda n: (0, 0)),
        ],
        out_specs=pl.BlockSpec((_B, 64), lambda n: (0, 0)),
        out_shape=jax.ShapeDtypeStruct((_B, 64), jnp.float32),
    )(acc2, y, deg, b16, gcn_b, mi_w, mi_b)


def _k3_body(p10_ref, p20_ref, p11_ref, p21_ref, p12_ref, p22_ref,
             mow0_ref, mob0_ref, mow1_ref, mob1_ref, mow2_ref, mob2_ref,
             csw0a_ref, csw0b_ref, csw0c_ref, csb0_ref,
             csw1_ref, csb1_ref, scw0_ref, scb0_ref, scw1_ref, scb1_ref,
             out_ref):
    def diff(p1_ref, p2_ref, mow_ref, mob_ref):
        o1 = jnp.maximum(
            jnp.dot(p1_ref[...], mow_ref[...],
                    preferred_element_type=jnp.float32) + mob_ref[...], 0.0)
        o2 = jnp.maximum(
            jnp.dot(p2_ref[...], mow_ref[...],
                    preferred_element_type=jnp.float32) + mob_ref[...], 0.0)
        return jnp.exp(-jnp.square(o1 - o2))

    d0 = diff(p10_ref, p20_ref, mow0_ref, mob0_ref)
    d1 = diff(p11_ref, p21_ref, mow1_ref, mob1_ref)
    d2 = diff(p12_ref, p22_ref, mow2_ref, mob2_ref)
    h = (jnp.dot(d0, csw0a_ref[...], preferred_element_type=jnp.float32)
         + jnp.dot(d1, csw0b_ref[...], preferred_element_type=jnp.float32)
         + jnp.dot(d2, csw0c_ref[...], preferred_element_type=jnp.float32)
         + csb0_ref[...])
    h = jnp.maximum(h, 0.0)
    h = jnp.tanh(
        jnp.dot(h, csw1_ref[...], preferred_element_type=jnp.float32)
        + csb1_ref[...])
    s = jnp.maximum(
        jnp.dot(h, scw0_ref[...], preferred_element_type=jnp.float32)
        + scb0_ref[...], 0.0)
    z = (jnp.dot(s, scw1_ref[...], preferred_element_type=jnp.float32)
         + scb1_ref[...])
    out_ref[...] = 1.0 / (1.0 + jnp.exp(-z))


def _tc_head(pooled, mo, cs_w0, cs_b0, cs_w1, cs_b1,
             sc_w0, sc_b0, sc_w1, sc_b1):
    args = [pooled[0][0], pooled[0][1], pooled[1][0], pooled[1][1],
            pooled[2][0], pooled[2][1],
            mo[0][0], mo[0][1].reshape(1, -1),
            mo[1][0], mo[1][1].reshape(1, -1),
            mo[2][0], mo[2][1].reshape(1, -1),
            cs_w0[0:64], cs_w0[64:128], cs_w0[128:192],
            cs_b0.reshape(1, -1),
            cs_w1, cs_b1.reshape(1, -1),
            sc_w0, sc_b0.reshape(1, -1),
            sc_w1, sc_b1.reshape(1, 1)]
    return pl.pallas_call(
        _k3_body,
        out_shape=jax.ShapeDtypeStruct((_B, 1), jnp.float32),
    )(*args)


def kernel(features_1, features_2, edge_index_1, edge_index_2,
           batch_1, batch_2,
           gcn_W0, gcn_b0, mi_W0, mi_b0, mo_W0, mo_b0,
           gcn_W1, gcn_b1, mi_W1, mi_b1, mo_W1, mo_b1,
           gcn_W2, gcn_b2, mi_W2, mi_b2, mo_W2, mo_b2,
           cs_W0, cs_b0, cs_W1, cs_b1, sc_W0, sc_b0, sc_W1, sc_b1):
    f32 = jnp.float32
    epad = _N + (jnp.arange(_EPAD - _E, dtype=jnp.int32) % (_NPAD - _N))
    src1 = jnp.concatenate([edge_index_1[0], epad]).reshape(_ERB, _CH)
    dst1 = jnp.concatenate([edge_index_1[1], epad]).reshape(_ERB, _CH)
    src2 = jnp.concatenate([edge_index_2[0], epad]).reshape(_ERB, _CH)
    dst2 = jnp.concatenate([edge_index_2[1], epad]).reshape(_ERB, _CH)
    dstcat = jnp.concatenate([dst1, dst2], axis=0)

    zeros16 = jnp.zeros((_NPAD, 16), f32)
    zeros64 = jnp.zeros((_NPAD, 64), f32)
    ones16 = jnp.ones((_CH, 16), f32)

    degcat = _sc_degree(dstcat, ones16, zeros16).reshape(2, _NPAD, 16)
    deg = [degcat[0], degcat[1]]

    bpad = jnp.full((_NPAD - _N,), _B, jnp.int32)
    b16 = [jnp.broadcast_to(
               jnp.concatenate([b, bpad]).astype(f32)[:, None], (_NPAD, 16))
           for b in (batch_1, batch_2)]

    x = [jnp.pad(features_1, ((0, _NPAD - _N), (0, 0))),
         jnp.pad(features_2, ((0, _NPAD - _N), (0, 0)))]
    src = [src1, src2]
    dst = [dst1, dst2]

    gcn = [(gcn_W0, gcn_b0.reshape(1, -1)),
           (gcn_W1, gcn_b1.reshape(1, -1)),
           (gcn_W2, gcn_b2.reshape(1, -1))]
    mi = [(mi_W0, mi_b0.reshape(1, -1)),
          (mi_W1, mi_b1.reshape(1, -1)),
          (mi_W2, mi_b2.reshape(1, -1))]
    mo = [(mo_W0, mo_b0), (mo_W1, mo_b1), (mo_W2, mo_b2)]

    y = [_tc_scale_matmul(x[0], gcn[0][0], deg[0]),
         _tc_scale_matmul(x[1], gcn[0][0], deg[1])]
    pooled = []
    for i in range(3):
        acc, p = [None, None], [None, None]
        for g in range(2):
            acc[g] = _sc_edge_scatter(y[g], src[g], dst[g], zeros64)
        if i < 2:
            for g in range(2):
                y[g], p[g] = _tc_update_pool_next(
                    acc[g].reshape(2, _NPAD, 64), y[g], deg[g], b16[g],
                    gcn[i][1], mi[i][0], mi[i][1], gcn[i + 1][0])
        else:
            for g in range(2):
                p[g] = _tc_update_pool_final(
                    acc[g].reshape(2, _NPAD, 64), y[g], deg[g], b16[g],
                    gcn[i][1], mi[i][0], mi[i][1])
        pooled.append(p)

    score = _tc_head(pooled, mo, cs_W0, cs_b0, cs_W1, cs_b1,
                     sc_W0, sc_b0, sc_W1, sc_b1)
    return score.reshape(-1)
